# baseline (speedup 1.0000x reference)
import jax
import jax.numpy as jnp
from jax.experimental import pallas as pl

_TB = 32


def _log_softmax(x):
    m = jnp.max(x, axis=-1, keepdims=True)
    s = x - m
    return s - jnp.log(jnp.sum(jnp.exp(s), axis=-1, keepdims=True))


def _prep_kernel(tn_ref, pn_ref, tn_out_ref, pn_rep_ref):
    tn_out_ref[...] = _log_softmax(tn_ref[...])
    pn = _log_softmax(pn_ref[...])
    v = jax.lax.broadcasted_iota(jnp.int32, pn.shape, 1)
    pn = jnp.where(v == 0, 0.0, pn)
    pn_rep_ref[...] = jnp.tile(pn, (_TB, 1))


def _add_kernel(tn_ref, pn_rep_ref, out_ref):
    U = pn_rep_ref.shape[0] // tn_ref.shape[0]
    out_ref[...] = pn_rep_ref[...] + jnp.repeat(tn_ref[...], U, axis=0)


def kernel(tn_out, pn_out):
    B, T, V = tn_out.shape
    _, U, _ = pn_out.shape
    Tb = _TB
    Sb = Tb * U
    tn_ls, pn_rep = pl.pallas_call(
        _prep_kernel,
        grid=(B,),
        in_specs=[
            pl.BlockSpec((None, T, V), lambda b: (b, 0, 0)),
            pl.BlockSpec((None, U, V), lambda b: (b, 0, 0)),
        ],
        out_specs=[
            pl.BlockSpec((None, T, V), lambda b: (b, 0, 0)),
            pl.BlockSpec((None, Sb, V), lambda b: (b, 0, 0)),
        ],
        out_shape=[
            jax.ShapeDtypeStruct((B, T, V), tn_out.dtype),
            jax.ShapeDtypeStruct((B, Sb, V), pn_out.dtype),
        ],
    )(tn_out, pn_out)

    out = pl.pallas_call(
        _add_kernel,
        grid=(B, T // Tb),
        in_specs=[
            pl.BlockSpec((None, Tb, V), lambda b, t: (b, t, 0)),
            pl.BlockSpec((None, Sb, V), lambda b, t: (b, 0, 0)),
        ],
        out_specs=pl.BlockSpec((None, Sb, V), lambda b, t: (b, t, 0)),
        out_shape=jax.ShapeDtypeStruct((B, T * U, V), tn_out.dtype),
    )(tn_ls, pn_rep)
    return out.reshape(B, T, U, V)


# speedup vs baseline: 2.0293x; 2.0293x over previous
import jax
import jax.numpy as jnp
from jax.experimental import pallas as pl
from jax.experimental.pallas import tpu as pltpu


def _log_softmax(x):
    m = jnp.max(x, axis=-1, keepdims=True)
    s = x - m
    return s - jnp.log(jnp.sum(jnp.exp(s), axis=-1, keepdims=True))


def _prep_kernel(tn_ref, pn_ref, tn_out_ref, pn_out_ref):
    tn_out_ref[...] = _log_softmax(tn_ref[...])
    pn = _log_softmax(pn_ref[...])
    v = jax.lax.broadcasted_iota(jnp.int32, pn.shape, 1)
    pn_out_ref[...] = jnp.where(v == 0, 0.0, pn)


def _add_kernel(tn_ref, pn_ref, out_ref):
    out_ref[...] = tn_ref[...][:, None, :] + pn_ref[...][None, :, :]


def kernel(tn_out, pn_out):
    B, T, V = tn_out.shape
    _, U, _ = pn_out.shape
    tn_ls, pn_ls = pl.pallas_call(
        _prep_kernel,
        grid=(B,),
        in_specs=[
            pl.BlockSpec((None, T, V), lambda b: (b, 0, 0)),
            pl.BlockSpec((None, U, V), lambda b: (b, 0, 0)),
        ],
        out_specs=[
            pl.BlockSpec((None, T, V), lambda b: (b, 0, 0)),
            pl.BlockSpec((None, U, V), lambda b: (b, 0, 0)),
        ],
        out_shape=[
            jax.ShapeDtypeStruct((B, T, V), tn_out.dtype),
            jax.ShapeDtypeStruct((B, U, V), pn_out.dtype),
        ],
        compiler_params=pltpu.CompilerParams(
            dimension_semantics=("parallel",),
        ),
    )(tn_out, pn_out)

    Tb = 32
    return pl.pallas_call(
        _add_kernel,
        grid=(B, T // Tb),
        in_specs=[
            pl.BlockSpec((None, Tb, V), lambda b, t: (b, t, 0)),
            pl.BlockSpec((None, U, V), lambda b, t: (b, 0, 0)),
        ],
        out_specs=pl.BlockSpec((None, Tb, U, V), lambda b, t: (b, t, 0, 0)),
        out_shape=jax.ShapeDtypeStruct((B, T, U, V), tn_out.dtype),
        compiler_params=pltpu.CompilerParams(
            dimension_semantics=("parallel", "parallel"),
        ),
    )(tn_ls, pn_ls)


# speedup vs baseline: 2.3195x; 1.1430x over previous
import jax
import jax.numpy as jnp
from jax.experimental import pallas as pl
from jax.experimental.pallas import tpu as pltpu


def _log_softmax(x):
    m = jnp.max(x, axis=-1, keepdims=True)
    s = x - m
    return s - jnp.log(jnp.sum(jnp.exp(s), axis=-1, keepdims=True))


def _prep_kernel(tn_ref, pn_ref, tn_out_ref, pn_out_ref):
    tn_out_ref[...] = _log_softmax(tn_ref[...])
    pn = _log_softmax(pn_ref[...])
    v = jax.lax.broadcasted_iota(jnp.int32, pn.shape, 1)
    pn_out_ref[...] = jnp.where(v == 0, 0.0, pn)


def _add_kernel(tn_ref, pn_ref, out_ref):
    out_ref[...] = tn_ref[...][:, None, :] + pn_ref[...][None, :, :]


def kernel(tn_out, pn_out):
    B, T, V = tn_out.shape
    _, U, _ = pn_out.shape
    tn_ls, pn_ls = pl.pallas_call(
        _prep_kernel,
        grid=(B,),
        in_specs=[
            pl.BlockSpec((None, T, V), lambda b: (b, 0, 0)),
            pl.BlockSpec((None, U, V), lambda b: (b, 0, 0)),
        ],
        out_specs=[
            pl.BlockSpec((None, T, V), lambda b: (b, 0, 0)),
            pl.BlockSpec((None, U, V), lambda b: (b, 0, 0)),
        ],
        out_shape=[
            jax.ShapeDtypeStruct((B, T, V), tn_out.dtype),
            jax.ShapeDtypeStruct((B, U, V), pn_out.dtype),
        ],
        compiler_params=pltpu.CompilerParams(
            dimension_semantics=("parallel",),
        ),
    )(tn_out, pn_out)

    Tb = 256
    return pl.pallas_call(
        _add_kernel,
        grid=(B, T // Tb),
        in_specs=[
            pl.BlockSpec((None, Tb, V), lambda b, t: (b, t, 0)),
            pl.BlockSpec((None, U, V), lambda b, t: (b, 0, 0)),
        ],
        out_specs=pl.BlockSpec((None, Tb, U, V), lambda b, t: (b, t, 0, 0)),
        out_shape=jax.ShapeDtypeStruct((B, T, U, V), tn_out.dtype),
        compiler_params=pltpu.CompilerParams(
            dimension_semantics=("parallel", "parallel"),
        ),
    )(tn_ls, pn_ls)
